# baseline (speedup 1.0000x reference)
import jax
import jax.numpy as jnp
from jax.experimental import pallas as pl
from jax.experimental.pallas import tpu as pltpu


def _support_body(x_ref, w_ref, o_ref):
    o_ref[:, :] = jnp.dot(x_ref[:, :], w_ref[:, :],
                          preferred_element_type=jnp.float32)


def _agg_body(a1_ref, a2_ref, s_ref, b_ref, o_ref, *, d_out):
    s1 = s_ref[:, :d_out]
    s2 = s_ref[:, d_out:]
    acc = jnp.dot(a1_ref[:, :], s1, preferred_element_type=jnp.float32)
    acc = acc + jnp.dot(a2_ref[:, :], s2, preferred_element_type=jnp.float32)
    o_ref[:, :] = acc + b_ref[:, :]


def kernel(input, adj_1, adj_2, weight_1, weight_2, bias):
    n, d_in = input.shape
    d_out = weight_1.shape[1]

    w = jnp.concatenate([weight_1, weight_2], axis=1)
    bms = 1000
    s = pl.pallas_call(
        _support_body,
        out_shape=jax.ShapeDtypeStruct((n, 2 * d_out), jnp.float32),
        grid=(n // bms,),
        in_specs=[
            pl.BlockSpec((bms, d_in), lambda i: (i, 0)),
            pl.BlockSpec((d_in, 2 * d_out), lambda i: (0, 0)),
        ],
        out_specs=pl.BlockSpec((bms, 2 * d_out), lambda i: (i, 0)),
    )(input, w)

    bm = 200
    bias2d = bias.reshape(1, d_out)
    import functools
    out = pl.pallas_call(
        functools.partial(_agg_body, d_out=d_out),
        out_shape=jax.ShapeDtypeStruct((n, d_out), jnp.float32),
        grid=(n // bm,),
        in_specs=[
            pl.BlockSpec((bm, n), lambda i: (i, 0)),
            pl.BlockSpec((bm, n), lambda i: (i, 0)),
            pl.BlockSpec((n, 2 * d_out), lambda i: (0, 0)),
            pl.BlockSpec((1, d_out), lambda i: (0, 0)),
        ],
        out_specs=pl.BlockSpec((bm, d_out), lambda i: (i, 0)),
        compiler_params=pltpu.CompilerParams(
            dimension_semantics=("parallel",),
        ),
    )(adj_1, adj_2, s, bias2d)
    return out


# speedup vs baseline: 1.0475x; 1.0475x over previous
import functools

import jax
import jax.numpy as jnp
from jax.experimental import pallas as pl
from jax.experimental.pallas import tpu as pltpu


def _body(x_ref, w_ref, a1_ref, a2_ref, b_ref, o_ref, s_ref, *, d_out):
    @pl.when(pl.program_id(0) == 0)
    def _():
        s_ref[:, :] = jnp.dot(x_ref[:, :], w_ref[:, :],
                              preferred_element_type=jnp.float32)

    s1 = s_ref[:, :d_out]
    s2 = s_ref[:, d_out:]
    acc = jnp.dot(a1_ref[:, :], s1, preferred_element_type=jnp.float32)
    acc = acc + jnp.dot(a2_ref[:, :], s2, preferred_element_type=jnp.float32)
    o_ref[:, :] = acc + b_ref[:, :]


def kernel(input, adj_1, adj_2, weight_1, weight_2, bias):
    n, d_in = input.shape
    d_out = weight_1.shape[1]

    w = jnp.concatenate([weight_1, weight_2], axis=1)
    bias2d = bias.reshape(1, d_out)
    bm = 200

    out = pl.pallas_call(
        functools.partial(_body, d_out=d_out),
        out_shape=jax.ShapeDtypeStruct((n, d_out), jnp.float32),
        grid=(n // bm,),
        in_specs=[
            pl.BlockSpec((n, d_in), lambda i: (0, 0)),
            pl.BlockSpec((d_in, 2 * d_out), lambda i: (0, 0)),
            pl.BlockSpec((bm, n), lambda i: (i, 0)),
            pl.BlockSpec((bm, n), lambda i: (i, 0)),
            pl.BlockSpec((1, d_out), lambda i: (0, 0)),
        ],
        out_specs=pl.BlockSpec((bm, d_out), lambda i: (i, 0)),
        scratch_shapes=[pltpu.VMEM((n, 2 * d_out), jnp.float32)],
        compiler_params=pltpu.CompilerParams(
            dimension_semantics=("arbitrary",),
        ),
    )(input, w, adj_1, adj_2, bias2d)
    return out
